# baseline (speedup 1.0000x reference)
import functools
import math

import jax
import jax.numpy as jnp
from jax import lax
from jax.experimental import pallas as pl
from jax.experimental.pallas import tpu as pltpu
from jax.experimental.pallas import tpu_sc as plsc

B = 4
SEQ = 2048
D_F = 1024
SCALE = math.sqrt(D_F)

_NC = plsc.get_sparse_core_info().num_cores

_mesh = plsc.VectorSubcoreMesh(core_axis_name="c", subcore_axis_name="s")


@functools.partial(
    pl.kernel,
    mesh=_mesh,
    out_type=jax.ShapeDtypeStruct((8, D_F), jnp.float32),
    scratch_types=[
        pltpu.VMEM((8,), jnp.int32),
        pltpu.VMEM((8, D_F), jnp.float32),
        pltpu.SemaphoreType.DMA,
    ],
)
def _sc_gather(table_hbm, idx_hbm, out_hbm, idx_v, rows_v, sem):
    wid = lax.axis_index("s") * _NC + lax.axis_index("c")

    @pl.when(wid == 0)
    def _():
        pltpu.sync_copy(idx_hbm, idx_v)
        pltpu.async_copy(table_hbm.at[idx_v], rows_v, sem).wait()
        pltpu.sync_copy(rows_v, out_hbm)


def _tc_body(x_ref, pe_ref, seg_ref, o_ref):
    o_ref[...] = x_ref[...] * SCALE + (pe_ref[...] + seg_ref[0])


@jax.jit
def _run(x, idx8, pe, seg_table):
    seg_rows = _sc_gather(seg_table, idx8)
    return pl.pallas_call(
        _tc_body,
        grid=(B,),
        in_specs=[
            pl.BlockSpec((1, SEQ, D_F), lambda b: (b, 0, 0)),
            pl.BlockSpec((1, SEQ, D_F), lambda b: (0, 0, 0)),
            pl.BlockSpec((8, D_F), lambda b: (0, 0)),
        ],
        out_specs=pl.BlockSpec((1, SEQ, D_F), lambda b: (b, 0, 0)),
        out_shape=jax.ShapeDtypeStruct((B, SEQ, D_F), jnp.float32),
    )(x, pe, seg_rows)


def kernel(x, view_idx, pe, seg_table):
    seq_len = x.shape[1]
    idx8 = jnp.full(
        (8,), jnp.asarray(view_idx, jnp.int32) * seq_len, dtype=jnp.int32
    )
    return _run(x, idx8, pe, seg_table)


# speedup vs baseline: 1.0245x; 1.0245x over previous
import functools
import math

import jax
import jax.numpy as jnp
from jax import lax
from jax.experimental import pallas as pl
from jax.experimental.pallas import tpu as pltpu
from jax.experimental.pallas import tpu_sc as plsc

B = 4
SEQ = 2048
D_F = 1024
SCALE = math.sqrt(D_F)

_NC = plsc.get_sparse_core_info().num_cores

_mesh = plsc.VectorSubcoreMesh(core_axis_name="c", subcore_axis_name="s")


@functools.partial(
    pl.kernel,
    mesh=_mesh,
    out_type=jax.ShapeDtypeStruct((1, D_F), jnp.float32),
    scratch_types=[
        pltpu.VMEM((1,), jnp.int32),
        pltpu.VMEM((1, D_F), jnp.float32),
        pltpu.SemaphoreType.DMA,
    ],
)
def _sc_gather(table_hbm, idx_hbm, out_hbm, idx_v, rows_v, sem):
    wid = lax.axis_index("s") * _NC + lax.axis_index("c")

    @pl.when(wid == 0)
    def _():
        pltpu.sync_copy(idx_hbm, idx_v)
        pltpu.async_copy(table_hbm.at[idx_v], rows_v, sem).wait()
        pltpu.sync_copy(rows_v, out_hbm)


def _tc_body(x_ref, pe_ref, seg_ref, o_ref):
    o_ref[...] = x_ref[...] * SCALE + (pe_ref[...] + seg_ref[0])


@jax.jit
def _run(x, idx8, pe, seg_table):
    seg_rows = _sc_gather(seg_table, idx8)
    return pl.pallas_call(
        _tc_body,
        grid=(B,),
        in_specs=[
            pl.BlockSpec((1, SEQ, D_F), lambda b: (b, 0, 0)),
            pl.BlockSpec((1, SEQ, D_F), lambda b: (0, 0, 0)),
            pl.BlockSpec((1, D_F), lambda b: (0, 0)),
        ],
        out_specs=pl.BlockSpec((1, SEQ, D_F), lambda b: (b, 0, 0)),
        out_shape=jax.ShapeDtypeStruct((B, SEQ, D_F), jnp.float32),
    )(x, pe, seg_rows)


def kernel(x, view_idx, pe, seg_table):
    seq_len = x.shape[1]
    idx1 = jnp.full(
        (1,), jnp.asarray(view_idx, jnp.int32) * seq_len, dtype=jnp.int32
    )
    return _run(x, idx1, pe, seg_table)
